# Initial kernel scaffold; baseline (speedup 1.0000x reference)
import jax
import jax.numpy as jnp
from jax.experimental import pallas as pl


def kernel(x, wq, bq, wk, bk, wv, bv, wo, bo, g1, be1, w1, b1, w2, b2, g2, be2):
    raise NotImplementedError("write your pallas kernel here")


# speedup vs baseline: 3.1077x; 3.1077x over previous
import math

import jax
import jax.numpy as jnp
from jax.experimental import pallas as pl
from jax.experimental.pallas import tpu as pltpu

_N_HEAD = 12
_D_HEAD = 64


def _encoder_kernel(x_ref, wq_ref, wk_ref, wv_ref, bq_ref, bk_ref, bv_ref,
                    wo_ref, bo_ref, g1_ref, be1_ref,
                    w1_ref, b1_ref, w2_ref, b2_ref, g2_ref, be2_ref,
                    out_ref):
    eps = 1e-12

    x = x_ref[0]
    xb = x.astype(jnp.bfloat16)

    q = jnp.dot(xb, wq_ref[...], preferred_element_type=jnp.float32) + bq_ref[...]
    k = jnp.dot(xb, wk_ref[...], preferred_element_type=jnp.float32) + bk_ref[...]
    v = jnp.dot(xb, wv_ref[...], preferred_element_type=jnp.float32) + bv_ref[...]
    qb = q.astype(jnp.bfloat16)
    kb = k.astype(jnp.bfloat16)
    vb = v.astype(jnp.bfloat16)

    outs = []
    for h in range(_N_HEAD):
        sl = slice(h * _D_HEAD, (h + 1) * _D_HEAD)
        s = jax.lax.dot_general(
            qb[:, sl], kb[:, sl], (((1,), (1,)), ((), ())),
            preferred_element_type=jnp.float32)
        m = jnp.max(s, axis=-1, keepdims=True)
        p = jnp.exp(s - m)
        l = jnp.sum(p, axis=-1, keepdims=True)
        o = jnp.dot(p.astype(jnp.bfloat16), vb[:, sl],
                    preferred_element_type=jnp.float32)
        outs.append(o / l)
    attn = jnp.concatenate(outs, axis=-1).astype(jnp.bfloat16)

    attn = jnp.dot(attn, wo_ref[...],
                   preferred_element_type=jnp.float32) + bo_ref[...]

    y = x + attn
    mu = jnp.mean(y, axis=-1, keepdims=True)
    yc = y - mu
    var = jnp.mean(yc * yc, axis=-1, keepdims=True)
    y = yc * jax.lax.rsqrt(var + eps)
    y = y * g1_ref[...] + be1_ref[...]

    h1 = jnp.dot(y.astype(jnp.bfloat16), w1_ref[...],
                 preferred_element_type=jnp.float32) + b1_ref[...]
    h1 = jnp.maximum(h1, 0.0).astype(jnp.bfloat16)
    f = jnp.dot(h1, w2_ref[...],
                preferred_element_type=jnp.float32) + b2_ref[...]

    z = y + f
    mu2 = jnp.mean(z, axis=-1, keepdims=True)
    zc = z - mu2
    var2 = jnp.mean(zc * zc, axis=-1, keepdims=True)
    z = zc * jax.lax.rsqrt(var2 + eps)
    z = z * g2_ref[...] + be2_ref[...]

    out_ref[0] = z.astype(out_ref.dtype)


def kernel(x, wq, bq, wk, bk, wv, bv, wo, bo, g1, be1,
           w1, b1, w2, b2, g2, be2):
    B, S, D = x.shape
    ffn_hidden = w1.shape[1]
    scale = 1.0 / math.sqrt(_D_HEAD)

    wq_b = (wq * scale).astype(jnp.bfloat16)
    wk_b = wk.astype(jnp.bfloat16)
    wv_b = wv.astype(jnp.bfloat16)
    wo_b = wo.astype(jnp.bfloat16)
    w1_b = w1.astype(jnp.bfloat16)
    w2_b = w2.astype(jnp.bfloat16)
    bq_s = bq * scale

    def fullb(shape):
        return pl.BlockSpec(shape, lambda b: (0,) * len(shape))

    out = pl.pallas_call(
        _encoder_kernel,
        out_shape=jax.ShapeDtypeStruct((B, S, D), x.dtype),
        grid_spec=pltpu.PrefetchScalarGridSpec(
            num_scalar_prefetch=0,
            grid=(B,),
            in_specs=[
                pl.BlockSpec((1, S, D), lambda b: (b, 0, 0)),
                fullb((D, D)),
                fullb((D, D)),
                fullb((D, D)),
                fullb((1, D)), fullb((1, D)), fullb((1, D)),
                fullb((D, D)), fullb((1, D)),
                fullb((1, D)), fullb((1, D)),
                fullb((D, ffn_hidden)), fullb((1, ffn_hidden)),
                fullb((ffn_hidden, D)), fullb((1, D)),
                fullb((1, D)), fullb((1, D)),
            ],
            out_specs=pl.BlockSpec((1, S, D), lambda b: (b, 0, 0)),
        ),
        compiler_params=pltpu.CompilerParams(
            dimension_semantics=("parallel",),
            vmem_limit_bytes=64 * 1024 * 1024),
    )(x, wq_b, wk_b, wv_b, bq_s, bk, bv, wo_b, bo, g1, be1,
      w1_b, b1, w2_b, b2, g2, be2)
    return out
